# Initial kernel scaffold; baseline (speedup 1.0000x reference)
import jax
import jax.numpy as jnp
from jax.experimental import pallas as pl


def kernel(token_ids, position_ids, segment_ids, tok_table, pos_table, seg_table):
    raise NotImplementedError("write your pallas kernel here")


# speedup vs baseline: 8.8497x; 8.8497x over previous
import functools

import jax
import jax.numpy as jnp
from jax import lax
from jax.experimental import pallas as pl
from jax.experimental.pallas import tpu as pltpu
from jax.experimental.pallas import tpu_sc as plsc

VOCAB = 100000
EMBED = 128
MAX_LEN = 512
NUM_SEG = 2
B, S = 4096, 200
N = B * S

_info = plsc.get_sparse_core_info()
NC, NS, L = _info.num_cores, _info.num_subcores, _info.num_lanes
NW = NC * NS
PER_TILE = N // NW
C = 128
CHUNKS = PER_TILE // C


def _sc_embed(tok_ids, cmb_ids, tok_table, cmb_table):
    mesh = plsc.VectorSubcoreMesh(core_axis_name="c", subcore_axis_name="s")

    @functools.partial(
        pl.kernel,
        mesh=mesh,
        out_type=jax.ShapeDtypeStruct((N, EMBED), jnp.float32),
        scratch_types=[
            pltpu.VMEM((C,), jnp.int32),
            pltpu.VMEM((C,), jnp.int32),
            pltpu.VMEM((C, EMBED), jnp.float32),
            pltpu.VMEM((C, EMBED), jnp.float32),
            pltpu.SemaphoreType.DMA,
            pltpu.SemaphoreType.DMA,
        ],
    )
    def k(tok_ids_hbm, cmb_ids_hbm, tok_tab_hbm, cmb_tab_hbm, out_hbm,
          idx_tok, idx_cmb, buf_a, buf_b, sem_a, sem_b):
        wid = lax.axis_index("s") * NC + lax.axis_index("c")
        tile_base = wid * PER_TILE

        def chunk_body(g, _):
            base = pl.multiple_of(tile_base + g * C, C)
            pltpu.sync_copy(tok_ids_hbm.at[pl.ds(base, C)], idx_tok)
            pltpu.sync_copy(cmb_ids_hbm.at[pl.ds(base, C)], idx_cmb)
            ga = pltpu.async_copy(tok_tab_hbm.at[idx_tok], buf_a, sem_a)
            gb = pltpu.async_copy(cmb_tab_hbm.at[idx_cmb], buf_b, sem_b)
            ga.wait()
            gb.wait()

            def add_body(r, _):
                for j in range(EMBED // L):
                    sl = pl.ds(j * L, L)
                    buf_a[r, sl] = buf_a[r, sl] + buf_b[r, sl]
                return 0

            lax.fori_loop(0, C, add_body, 0)
            pltpu.sync_copy(buf_a, out_hbm.at[pl.ds(base, C)])
            return 0

        lax.fori_loop(0, CHUNKS, chunk_body, 0)

    return k(tok_ids, cmb_ids, tok_table, cmb_table)


def kernel(token_ids, position_ids, segment_ids, tok_table, pos_table, seg_table):
    tok_flat = token_ids.reshape(N).astype(jnp.int32)
    cmb_flat = (position_ids.reshape(N) * NUM_SEG + segment_ids.reshape(N)).astype(jnp.int32)
    cmb_table = (pos_table[:, None, :] + seg_table[None, :, :]).reshape(
        MAX_LEN * NUM_SEG, EMBED)
    out = _sc_embed(tok_flat, cmb_flat, tok_table, cmb_table)
    return out.reshape(B, S, EMBED)


# speedup vs baseline: 14.9583x; 1.6903x over previous
import functools

import jax
import jax.numpy as jnp
from jax import lax
from jax.experimental import pallas as pl
from jax.experimental.pallas import tpu as pltpu
from jax.experimental.pallas import tpu_sc as plsc

VOCAB = 100000
EMBED = 128
MAX_LEN = 512
NUM_SEG = 2
B, S = 4096, 200
N = B * S

_info = plsc.get_sparse_core_info()
NC, NS, L = _info.num_cores, _info.num_subcores, _info.num_lanes
NW = NC * NS
PER_TILE = N // NW
C = 128
CHUNKS = PER_TILE // C
NBUF = 2
OUTER = CHUNKS // NBUF


def _sc_embed(tok_ids, cmb_ids, tok_table, cmb_table):
    mesh = plsc.VectorSubcoreMesh(core_axis_name="c", subcore_axis_name="s")

    @functools.partial(
        pl.kernel,
        mesh=mesh,
        out_type=jax.ShapeDtypeStruct((N, EMBED), jnp.float32),
        scratch_types=(
            [pltpu.VMEM((C,), jnp.int32) for _ in range(NBUF)]
            + [pltpu.VMEM((C,), jnp.int32) for _ in range(NBUF)]
            + [pltpu.VMEM((C, EMBED), jnp.float32) for _ in range(NBUF)]
            + [pltpu.VMEM((C, EMBED), jnp.float32) for _ in range(NBUF)]
            + [pltpu.VMEM((C, EMBED), jnp.float32) for _ in range(NBUF)]
            + [pltpu.SemaphoreType.DMA for _ in range(3 * NBUF)]
        ),
    )
    def k(tok_ids_hbm, cmb_ids_hbm, tok_tab_hbm, cmb_tab_hbm, out_hbm, *scr):
        idx_tok = scr[0:NBUF]
        idx_cmb = scr[NBUF:2 * NBUF]
        buf_a = scr[2 * NBUF:3 * NBUF]
        buf_b = scr[3 * NBUF:4 * NBUF]
        buf_o = scr[4 * NBUF:5 * NBUF]
        sem_a = scr[5 * NBUF:6 * NBUF]
        sem_b = scr[6 * NBUF:7 * NBUF]
        sem_o = scr[7 * NBUF:8 * NBUF]

        wid = lax.axis_index("s") * NC + lax.axis_index("c")
        tile_base = wid * PER_TILE

        def issue(g, b):
            base = pl.multiple_of(tile_base + g * C, C)
            pltpu.sync_copy(tok_ids_hbm.at[pl.ds(base, C)], idx_tok[b])
            pltpu.sync_copy(cmb_ids_hbm.at[pl.ds(base, C)], idx_cmb[b])
            pltpu.async_copy(tok_tab_hbm.at[idx_tok[b]], buf_a[b], sem_a[b])
            pltpu.async_copy(cmb_tab_hbm.at[idx_cmb[b]], buf_b[b], sem_b[b])

        def wait_gathers(b):
            pltpu.make_async_copy(tok_tab_hbm.at[idx_tok[b]], buf_a[b],
                                  sem_a[b]).wait()
            pltpu.make_async_copy(cmb_tab_hbm.at[idx_cmb[b]], buf_b[b],
                                  sem_b[b]).wait()

        def drain_store(g, b):
            base = pl.multiple_of(tile_base + g * C, C)
            pltpu.make_async_copy(buf_o[b], out_hbm.at[pl.ds(base, C)],
                                  sem_o[b]).wait()

        for b in range(NBUF):
            issue(b, b)

        def outer_body(gp, _):
            for b in range(NBUF):
                g = gp * NBUF + b
                @pl.when(gp >= 1)
                def _():
                    drain_store(g - NBUF, b)
                wait_gathers(b)

                def add_body(r, _):
                    for j in range(EMBED // L):
                        sl = pl.ds(j * L, L)
                        buf_o[b][r, sl] = buf_a[b][r, sl] + buf_b[b][r, sl]
                    return 0

                lax.fori_loop(0, C, add_body, 0)
                base = pl.multiple_of(tile_base + g * C, C)
                pltpu.async_copy(buf_o[b], out_hbm.at[pl.ds(base, C)], sem_o[b])

                @pl.when(gp + 1 < OUTER)
                def _():
                    issue(g + NBUF, b)
            return 0

        lax.fori_loop(0, OUTER, outer_body, 0)
        for b in range(NBUF):
            drain_store(CHUNKS - NBUF + b, b)

    return k(tok_ids, cmb_ids, tok_table, cmb_table)


def kernel(token_ids, position_ids, segment_ids, tok_table, pos_table, seg_table):
    tok_flat = token_ids.reshape(N).astype(jnp.int32)
    cmb_flat = (position_ids.reshape(N) * NUM_SEG + segment_ids.reshape(N)).astype(jnp.int32)
    cmb_table = (pos_table[:, None, :] + seg_table[None, :, :]).reshape(
        MAX_LEN * NUM_SEG, EMBED)
    out = _sc_embed(tok_flat, cmb_flat, tok_table, cmb_table)
    return out.reshape(B, S, EMBED)
